# baseline (speedup 1.0000x reference)
import functools

import jax
import jax.numpy as jnp
from jax import lax
from jax.experimental import pallas as pl
from jax.experimental.pallas import tpu as pltpu
from jax.experimental.pallas import tpu_sc as plsc

_SIZE = 1_000_000
_DIM = 64
_BATCH = 16384
_NW = 32
_PER_W = _BATCH // _NW
_GRP = 16

_NCHUNK = 8
_ROWS_PER_CHUNK = _SIZE // _NCHUNK


def _copy_body(src, dst, sems):
    for c in range(_NCHUNK):
        pltpu.make_async_copy(
            src.at[pl.ds(c * _ROWS_PER_CHUNK, _ROWS_PER_CHUNK)],
            dst.at[pl.ds(c * _ROWS_PER_CHUNK, _ROWS_PER_CHUNK)],
            sems.at[c],
        ).start()
    for c in range(_NCHUNK):
        pltpu.make_async_copy(
            src.at[pl.ds(c * _ROWS_PER_CHUNK, _ROWS_PER_CHUNK)],
            dst.at[pl.ds(c * _ROWS_PER_CHUNK, _ROWS_PER_CHUNK)],
            sems.at[c],
        ).wait()


_copy = pl.pallas_call(
    _copy_body,
    out_shape=jax.ShapeDtypeStruct((_SIZE, _DIM), jnp.float32),
    in_specs=[pl.BlockSpec(memory_space=pl.ANY)],
    out_specs=pl.BlockSpec(memory_space=pl.ANY),
    scratch_shapes=[pltpu.SemaphoreType.DMA((_NCHUNK,))],
)

_mesh = plsc.VectorSubcoreMesh(core_axis_name="c", subcore_axis_name="s")


@functools.partial(
    pl.kernel,
    mesh=_mesh,
    out_type=(),
    scratch_types=[
        pltpu.VMEM((_PER_W,), jnp.int32),
        pltpu.VMEM((_PER_W,), jnp.int32),
        pltpu.VMEM((_PER_W, _DIM), jnp.float32),
        pltpu.SemaphoreType.DMA,
        pltpu.SemaphoreType.DMA,
    ],
)
def _sc_scatter(out_ref, tgt_hbm, src_hbm, data_hbm,
                tgt_v, src_v, rows_v, g_sem, s_sem):
    wid = lax.axis_index("s") * 2 + lax.axis_index("c")
    base = wid * _PER_W
    pltpu.sync_copy(tgt_hbm.at[pl.ds(base, _PER_W)], tgt_v)
    pltpu.sync_copy(src_hbm.at[pl.ds(base, _PER_W)], src_v)

    def fire_gathers(g, carry):
        svec = src_v[pl.ds(g * _GRP, _GRP)]
        for lane in range(_GRP):
            k = g * _GRP + lane
            pltpu.async_copy(
                data_hbm.at[pl.ds(svec[lane], 1)],
                rows_v.at[pl.ds(k, 1)],
                g_sem,
            )
        return carry

    lax.fori_loop(0, _PER_W // _GRP, fire_gathers, 0)
    pltpu.make_async_copy(data_hbm.at[pl.ds(0, _PER_W)], rows_v, g_sem).wait()

    def fire_scatters(g, carry):
        tvec = tgt_v[pl.ds(g * _GRP, _GRP)]
        for lane in range(_GRP):
            k = g * _GRP + lane
            pltpu.async_copy(
                rows_v.at[pl.ds(k, 1)],
                out_ref.at[pl.ds(tvec[lane], 1)],
                s_sem,
            )
        return carry

    lax.fori_loop(0, _PER_W // _GRP, fire_scatters, 0)
    pltpu.make_async_copy(data_hbm.at[pl.ds(0, _PER_W)], rows_v, s_sem).wait()


def kernel(bank, indices, data_memory):
    order = jnp.argsort(indices, stable=True).astype(jnp.int32)
    si = jnp.take(indices, order)
    islast = jnp.concatenate([si[:-1] != si[1:], jnp.ones((1,), jnp.bool_)])
    slot = jnp.arange(_BATCH, dtype=jnp.int32)
    cand = jnp.where(islast, slot, _BATCH)
    last_slot = lax.cummin(cand, reverse=True)
    src_pos = jnp.take(order, last_slot).astype(jnp.int32)

    new_bank = _copy(bank)
    ref = jax.new_ref(new_bank)
    _sc_scatter(ref, si, src_pos, data_memory)
    return jax.freeze(ref)


# speedup vs baseline: 15.4236x; 15.4236x over previous
import functools

import jax
import jax.numpy as jnp
from jax import lax
from jax.experimental import pallas as pl
from jax.experimental.pallas import tpu as pltpu
from jax.experimental.pallas import tpu_sc as plsc

_SIZE = 1_000_000
_DIM = 64
_BATCH = 16384
_NW = 32
_PER_W = _BATCH // _NW
_GRP = 16

_BLK = 8000
_GRID = _SIZE // _BLK


def _copy_body(src, dst):
    dst[...] = src[...]


_copy = pl.pallas_call(
    _copy_body,
    grid=(_GRID,),
    in_specs=[pl.BlockSpec((_BLK, _DIM), lambda i: (i, 0))],
    out_specs=pl.BlockSpec((_BLK, _DIM), lambda i: (i, 0)),
    out_shape=jax.ShapeDtypeStruct((_SIZE, _DIM), jnp.float32),
    compiler_params=pltpu.CompilerParams(
        dimension_semantics=("arbitrary",),
    ),
)

_mesh = plsc.VectorSubcoreMesh(core_axis_name="c", subcore_axis_name="s")


@functools.partial(
    pl.kernel,
    mesh=_mesh,
    out_type=(),
    scratch_types=[
        pltpu.VMEM((_PER_W,), jnp.int32),
        pltpu.VMEM((_PER_W,), jnp.int32),
        pltpu.VMEM((_PER_W, _DIM), jnp.float32),
        pltpu.SemaphoreType.DMA,
        pltpu.SemaphoreType.DMA,
    ],
)
def _sc_scatter(out_ref, tgt_hbm, src_hbm, data_hbm,
                tgt_v, src_v, rows_v, g_sem, s_sem):
    wid = lax.axis_index("s") * 2 + lax.axis_index("c")
    base = wid * _PER_W
    pltpu.sync_copy(tgt_hbm.at[pl.ds(base, _PER_W)], tgt_v)
    pltpu.sync_copy(src_hbm.at[pl.ds(base, _PER_W)], src_v)

    def fire_gathers(g, carry):
        svec = src_v[pl.ds(g * _GRP, _GRP)]
        for lane in range(_GRP):
            k = g * _GRP + lane
            pltpu.async_copy(
                data_hbm.at[pl.ds(svec[lane], 1)],
                rows_v.at[pl.ds(k, 1)],
                g_sem,
            )
        return carry

    lax.fori_loop(0, _PER_W // _GRP, fire_gathers, 0)
    pltpu.make_async_copy(data_hbm.at[pl.ds(0, _PER_W)], rows_v, g_sem).wait()

    def fire_scatters(g, carry):
        tvec = tgt_v[pl.ds(g * _GRP, _GRP)]
        for lane in range(_GRP):
            k = g * _GRP + lane
            pltpu.async_copy(
                rows_v.at[pl.ds(k, 1)],
                out_ref.at[pl.ds(tvec[lane], 1)],
                s_sem,
            )
        return carry

    lax.fori_loop(0, _PER_W // _GRP, fire_scatters, 0)
    pltpu.make_async_copy(data_hbm.at[pl.ds(0, _PER_W)], rows_v, s_sem).wait()


def kernel(bank, indices, data_memory):
    order = jnp.argsort(indices, stable=True).astype(jnp.int32)
    si = jnp.take(indices, order)
    islast = jnp.concatenate([si[:-1] != si[1:], jnp.ones((1,), jnp.bool_)])
    slot = jnp.arange(_BATCH, dtype=jnp.int32)
    cand = jnp.where(islast, slot, _BATCH)
    last_slot = lax.cummin(cand, reverse=True)
    src_pos = jnp.take(order, last_slot).astype(jnp.int32)

    new_bank = _copy(bank)
    ref = jax.new_ref(new_bank)
    _sc_scatter(ref, si, src_pos, data_memory)
    return jax.freeze(ref)


# speedup vs baseline: 22.1581x; 1.4366x over previous
import functools

import jax
import jax.numpy as jnp
from jax import lax
from jax.experimental import pallas as pl
from jax.experimental.pallas import tpu as pltpu
from jax.experimental.pallas import tpu_sc as plsc

_SIZE = 1_000_000
_DIM = 64
_BATCH = 16384
_NW = 32
_PER_W = _BATCH // _NW
_GRP = 16

_BLK = 8000
_GRID = _SIZE // _BLK


def _copy_body(src, dst):
    dst[...] = src[...]


_copy = pl.pallas_call(
    _copy_body,
    grid=(_GRID,),
    in_specs=[pl.BlockSpec((_BLK, _DIM), lambda i: (i, 0))],
    out_specs=pl.BlockSpec((_BLK, _DIM), lambda i: (i, 0)),
    out_shape=jax.ShapeDtypeStruct((_SIZE, _DIM), jnp.float32),
    compiler_params=pltpu.CompilerParams(
        dimension_semantics=("arbitrary",),
    ),
)

_mesh = plsc.VectorSubcoreMesh(core_axis_name="c", subcore_axis_name="s")


@functools.partial(
    pl.kernel,
    mesh=_mesh,
    out_type=(),
    scratch_types=[
        pltpu.VMEM((_PER_W,), jnp.int32),
        pltpu.VMEM((_PER_W,), jnp.int32),
        pltpu.VMEM((_PER_W, _DIM), jnp.float32),
        pltpu.SemaphoreType.DMA,
        pltpu.SemaphoreType.DMA,
    ],
)
def _sc_scatter(out_ref, tgt_hbm, src_hbm, data_hbm,
                tgt_v, src_v, rows_v, g_sem, s_sem):
    wid = lax.axis_index("s") * 2 + lax.axis_index("c")
    base = wid * _PER_W
    pltpu.sync_copy(tgt_hbm.at[pl.ds(base, _PER_W)], tgt_v)
    pltpu.sync_copy(src_hbm.at[pl.ds(base, _PER_W)], src_v)

    def fire_gathers(g, carry):
        svec = src_v[pl.ds(g * _GRP, _GRP)]
        for lane in range(_GRP):
            k = g * _GRP + lane
            pltpu.async_copy(
                data_hbm.at[pl.ds(svec[lane], 1)],
                rows_v.at[pl.ds(k, 1)],
                g_sem,
            )
        return carry

    lax.fori_loop(0, _PER_W // _GRP, fire_gathers, 0)
    pltpu.make_async_copy(data_hbm.at[pl.ds(0, _PER_W)], rows_v, g_sem).wait()

    def fire_scatters(g, carry):
        tvec = tgt_v[pl.ds(g * _GRP, _GRP)]
        for lane in range(_GRP):
            k = g * _GRP + lane
            pltpu.async_copy(
                rows_v.at[pl.ds(k, 1)],
                out_ref.at[pl.ds(tvec[lane], 1)],
                s_sem,
            )
        return carry

    lax.fori_loop(0, _PER_W // _GRP, fire_scatters, 0)
    pltpu.make_async_copy(data_hbm.at[pl.ds(0, _PER_W)], rows_v, s_sem).wait()


def kernel(bank, indices, data_memory):
    order = jnp.argsort(indices, stable=True).astype(jnp.int32)
    si = jnp.take(indices, order)
    islast = jnp.concatenate([si[:-1] != si[1:], jnp.ones((1,), jnp.bool_)])
    slot = jnp.arange(_BATCH, dtype=jnp.int32)
    cand = jnp.where(islast, slot, _BATCH)
    last_slot = lax.cummin(cand, reverse=True)
    src_pos = jnp.take(order, last_slot).astype(jnp.int32)

    ref = jax.new_ref(bank)
    _sc_scatter(ref, si, src_pos, data_memory)
    return jax.freeze(ref)
